# baseline (speedup 1.0000x reference)
import functools

import jax
import jax.numpy as jnp
from jax import lax
from jax.experimental import pallas as pl
from jax.experimental.pallas import tpu as pltpu
from jax.experimental.pallas import tpu_sc as plsc
from jax._src.pallas import mpmd as _mpmd

NCLS = 1000
NROWS = 4096 * 26
NOUT = NROWS * NCLS
NCORES = 2
OUT_PER_CORE = NOUT // NCORES
ZIN = 1664000

NW = 32
ROWS_PER_W = NROWS // NW
LANES = 16
IDX_MINOR = 128
NIDX = ROWS_PER_W // IDX_MINOR

_smesh = plsc.ScalarSubcoreMesh(axis_name="c", num_cores=NCORES)
_vmesh = plsc.VectorSubcoreMesh(core_axis_name="c", subcore_axis_name="s")


@functools.partial(
    pl.kernel,
    out_type=jax.ShapeDtypeStruct((NOUT,), jnp.float32),
    mesh=_smesh,
    scratch_types=[pltpu.SemaphoreType.DMA],
)
def _sc_zero(zin_hbm, out_hbm, dsem):
    core = lax.axis_index("c")
    base = core * OUT_PER_CORE

    def zdma(i, carry):
        pltpu.async_copy(
            zin_hbm, out_hbm.at[pl.ds(base + i * ZIN, ZIN)], dsem
        )
        return carry

    lax.fori_loop(0, OUT_PER_CORE // ZIN, zdma, 0)

    def zwait(i, carry):
        pltpu.make_async_copy(
            zin_hbm, out_hbm.at[pl.ds(0, ZIN)], dsem
        ).wait()
        return carry

    lax.fori_loop(0, OUT_PER_CORE // ZIN, zwait, 0)


def _scatter_body(x_hbm, z_hbm, out_hbm, idx_v, off_v, ones_v, ssem):
    del z_hbm
    wid = lax.axis_index("s") * 2 + lax.axis_index("c")
    row0 = wid * ROWS_PER_W
    pltpu.sync_copy(x_hbm.at[pl.ds(row0, ROWS_PER_W)], idx_v)

    for k in range(IDX_MINOR // LANES):
        ones_v[pl.ds(k * LANES, LANES)] = jnp.ones((LANES,), jnp.float32)

    lane_iota = lax.iota(jnp.int32, LANES)

    def offs(j, carry):
        for k in range(IDX_MINOR // LANES):
            ids = idx_v[pl.ds(j * IDX_MINOR + k * LANES, LANES)]
            off_v[j, pl.ds(k * LANES, LANES)] = (
                (lane_iota + row0 + j * IDX_MINOR + k * LANES) * NCLS + ids
            )
        return carry

    lax.fori_loop(0, NIDX, offs, 0)

    def sdma(j, carry):
        pltpu.async_copy(ones_v, out_hbm.at[off_v.at[j]], ssem)
        return carry

    lax.fori_loop(0, NIDX, sdma, 0)

    def swait(j, carry):
        pltpu.make_async_copy(
            ones_v, out_hbm.at[pl.ds(0, IDX_MINOR)], ssem
        ).wait()
        return carry

    lax.fori_loop(0, NIDX, swait, 0)


_sc_scatter = _mpmd._mpmd_map(
    [(_vmesh, _scatter_body)],
    jax.ShapeDtypeStruct((NOUT,), jnp.float32),
    input_output_aliases={1: 0},
    scratch_types=[
        pltpu.VMEM((ROWS_PER_W,), jnp.int32),
        pltpu.VMEM((NIDX, IDX_MINOR), jnp.int32),
        pltpu.VMEM((IDX_MINOR,), jnp.float32),
        pltpu.SemaphoreType.DMA,
    ],
    compiler_params=pltpu.CompilerParams(needs_layout_passes=False),
)


def kernel(x):
    xf = x.reshape(-1).astype(jnp.int32)
    zin = jnp.zeros((ZIN,), jnp.float32)
    z = _sc_zero(zin)
    out = _sc_scatter(xf, z)
    return out.reshape(tuple(x.shape) + (NCLS,))


# speedup vs baseline: 10.2131x; 10.2131x over previous
import functools

import jax
import jax.numpy as jnp
from jax import lax
from jax.experimental import pallas as pl
from jax.experimental.pallas import tpu as pltpu
from jax.experimental.pallas import tpu_sc as plsc
from jax._src.pallas import mpmd as _mpmd

NCLS = 1000
NROWS = 4096 * 26
NOUT = NROWS * NCLS
NCORES = 2
OUT_PER_CORE = NOUT // NCORES
ZIN = 1664000

NW = 32
ROWS_PER_W = NROWS // NW
LANES = 16
IDX_MINOR = 128
NIDX = ROWS_PER_W // IDX_MINOR

_vmesh = plsc.VectorSubcoreMesh(core_axis_name="c", subcore_axis_name="s")

ZBLK = 512 * NCLS


def _zero_block(o_ref):
    o_ref[...] = jnp.zeros((ZBLK,), jnp.float32)


def _tc_zero():
    return pl.pallas_call(
        _zero_block,
        grid=(NOUT // ZBLK,),
        out_specs=pl.BlockSpec((ZBLK,), lambda i: (i,)),
        out_shape=jax.ShapeDtypeStruct((NOUT,), jnp.float32),
    )()


def _scatter_body(x_hbm, z_hbm, out_hbm, idx_v, off_v, ones_v, ssem):
    del z_hbm
    wid = lax.axis_index("s") * 2 + lax.axis_index("c")
    row0 = wid * ROWS_PER_W
    pltpu.sync_copy(x_hbm.at[pl.ds(row0, ROWS_PER_W)], idx_v)

    for k in range(IDX_MINOR // LANES):
        ones_v[pl.ds(k * LANES, LANES)] = jnp.ones((LANES,), jnp.float32)

    lane_iota = lax.iota(jnp.int32, LANES)

    def offs(j, carry):
        for k in range(IDX_MINOR // LANES):
            ids = idx_v[pl.ds(j * IDX_MINOR + k * LANES, LANES)]
            off_v[j, pl.ds(k * LANES, LANES)] = (
                (lane_iota + row0 + j * IDX_MINOR + k * LANES) * NCLS + ids
            )
        return carry

    lax.fori_loop(0, NIDX, offs, 0)

    def sdma(j, carry):
        pltpu.async_copy(ones_v, out_hbm.at[off_v.at[j]], ssem)
        return carry

    lax.fori_loop(0, NIDX, sdma, 0)

    def swait(j, carry):
        pltpu.make_async_copy(
            ones_v, out_hbm.at[pl.ds(0, IDX_MINOR)], ssem
        ).wait()
        return carry

    lax.fori_loop(0, NIDX, swait, 0)


_sc_scatter = _mpmd._mpmd_map(
    [(_vmesh, _scatter_body)],
    jax.ShapeDtypeStruct((NOUT,), jnp.float32),
    input_output_aliases={1: 0},
    scratch_types=[
        pltpu.VMEM((ROWS_PER_W,), jnp.int32),
        pltpu.VMEM((NIDX, IDX_MINOR), jnp.int32),
        pltpu.VMEM((IDX_MINOR,), jnp.float32),
        pltpu.SemaphoreType.DMA,
    ],
    compiler_params=pltpu.CompilerParams(needs_layout_passes=False),
)


def kernel(x):
    xf = x.reshape(-1).astype(jnp.int32)
    z = _tc_zero()
    out = _sc_scatter(xf, z)
    return out.reshape(tuple(x.shape) + (NCLS,))


# speedup vs baseline: 16.3996x; 1.6057x over previous
import functools

import jax
import jax.numpy as jnp
from jax import lax
from jax.experimental import pallas as pl
from jax.experimental.pallas import tpu as pltpu
from jax.experimental.pallas import tpu_sc as plsc

NCLS = 1000
NROWS = 4096 * 26
NW = 32
ROWS_PER_W = NROWS // NW
CHUNK_ROWS = 32
NCHUNK = ROWS_PER_W // CHUNK_ROWS
NBUF = 2
LANES = 16

_mesh = plsc.VectorSubcoreMesh(core_axis_name="c", subcore_axis_name="s")


@functools.partial(
    pl.kernel,
    out_type=jax.ShapeDtypeStruct((NROWS, NCLS), jnp.float32),
    mesh=_mesh,
    scratch_types=[
        pltpu.VMEM((ROWS_PER_W,), jnp.int32),
        pltpu.VMEM((NBUF * CHUNK_ROWS, NCLS), jnp.float32),
        pltpu.VMEM((NBUF * CHUNK_ROWS,), jnp.int32),
        pltpu.SemaphoreType.DMA,
        pltpu.SemaphoreType.DMA,
    ],
    compiler_params=pltpu.CompilerParams(needs_layout_passes=False),
)
def _sc_onehot(x_hbm, out_hbm, idx_v, buf, colbuf, sem0, sem1):
    wid = lax.axis_index("s") * 2 + lax.axis_index("c")
    row0 = wid * ROWS_PER_W
    pltpu.sync_copy(x_hbm.at[pl.ds(row0, ROWS_PER_W)], idx_v)

    zeros = jnp.zeros((LANES,), jnp.float32)
    ones = jnp.ones((LANES,), jnp.float32)
    lane_iota = lax.iota(jnp.int32, LANES)
    tail_mask = lane_iota < (NCLS % LANES)

    def zrow(r, carry):
        for c in range(NCLS // LANES):
            buf[r, pl.ds(c * LANES, LANES)] = zeros
        plsc.store_scatter(
            buf,
            [jnp.full((LANES,), r, jnp.int32), (NCLS // LANES) * LANES + lane_iota],
            zeros,
            mask=tail_mask,
        )
        return carry

    lax.fori_loop(0, NBUF * CHUNK_ROWS, zrow, 0)

    sems = (sem0, sem1)

    def outer(g, carry):
        for b in range(NBUF):
            c = g * NBUF + b

            @pl.when(g >= 1)
            def _wait_and_clear():
                pltpu.make_async_copy(
                    buf.at[pl.ds(b * CHUNK_ROWS, CHUNK_ROWS)],
                    out_hbm.at[pl.ds(0, CHUNK_ROWS)],
                    sems[b],
                ).wait()
                for j in range(CHUNK_ROWS // LANES):
                    rows = b * CHUNK_ROWS + j * LANES + lane_iota
                    old = colbuf[pl.ds(b * CHUNK_ROWS + j * LANES, LANES)]
                    plsc.store_scatter(buf, [rows, old], zeros)

            for j in range(CHUNK_ROWS // LANES):
                rows = b * CHUNK_ROWS + j * LANES + lane_iota
                ids = idx_v[pl.ds(c * CHUNK_ROWS + j * LANES, LANES)]
                plsc.store_scatter(buf, [rows, ids], ones)
                colbuf[pl.ds(b * CHUNK_ROWS + j * LANES, LANES)] = ids

            pltpu.async_copy(
                buf.at[pl.ds(b * CHUNK_ROWS, CHUNK_ROWS)],
                out_hbm.at[pl.ds(row0 + c * CHUNK_ROWS, CHUNK_ROWS)],
                sems[b],
            )
        return carry

    lax.fori_loop(0, NCHUNK // NBUF, outer, 0)

    for b in range(NBUF):
        pltpu.make_async_copy(
            buf.at[pl.ds(b * CHUNK_ROWS, CHUNK_ROWS)],
            out_hbm.at[pl.ds(0, CHUNK_ROWS)],
            sems[b],
        ).wait()


def kernel(x):
    xf = x.reshape(-1).astype(jnp.int32)
    out = _sc_onehot(xf)
    return out.reshape(tuple(x.shape) + (NCLS,))
